# baseline (speedup 1.0000x reference)
import functools

import jax
import jax.numpy as jnp
from jax import lax
from jax.experimental import pallas as pl
from jax.experimental.pallas import tpu as pltpu
from jax.experimental.pallas import tpu_sc as plsc

_B, _H, _S, _D, _Q = 8, 16, 4096, 128, 8
_BH = _B * _H
_BHB = 8
_W = 8
_NW = 32
_BH_PW = _BH // _NW
_SP = _S // 2
_CS = 512
_NCH = _BH_PW * (_S // _CS)
_NBUF = 3
_LOW = 65535
_HIGH = -65536

_mesh = plsc.VectorSubcoreMesh(core_axis_name="c", subcore_axis_name="s")


def _tc_kernel(pos_ref, c_ref, v_ref, o_ref):
    o_ref[...] = c_ref[...]
    sub_iota = jax.lax.broadcasted_iota(jnp.int32, (1, _W, 1), 1)
    for q in range(_Q):
        pos = pos_ref[q]
        wb = pl.multiple_of((pos // _W) * _W, _W)
        r = pos % _W
        mask = sub_iota == r
        o_ref[:, pl.ds(wb, _W), :] = jnp.where(
            mask, v_ref[:, q:q + 1, :], o_ref[:, pl.ds(wb, _W), :])


def _tc_update(cache, vals, input_pos):
    c = cache.reshape(_BH, _S, _D)
    v = vals.reshape(_BH, _Q, _D)
    cache_spec = pl.BlockSpec((_BHB, _S, _D), lambda i: (i, 0, 0))
    val_spec = pl.BlockSpec((_BHB, _Q, _D), lambda i: (i, 0, 0))
    out = pl.pallas_call(
        _tc_kernel,
        grid=(_BH // _BHB,),
        in_specs=[
            pl.BlockSpec(memory_space=pltpu.SMEM),
            cache_spec, val_spec,
        ],
        out_specs=cache_spec,
        out_shape=jax.ShapeDtypeStruct((_BH, _S, _D), c.dtype),
    )(input_pos, c, v)
    return out.reshape(_B, _H, _S, _D)


def _merge_rows(buf_p, buf_u, posbit16, qbit):
    def body(j, _):
        for c in range(_D // 16):
            o = buf_p[j, pl.ds(c * 16, 16)]
            u = buf_u[j, pl.ds(c * 16, 16)]
            if qbit == 0:
                val = jnp.bitwise_and(u, _LOW)
            else:
                val = lax.shift_right_logical(u, 16)
            lo_merged = jnp.bitwise_or(jnp.bitwise_and(o, _HIGH), val)
            hi_merged = jnp.bitwise_or(jnp.bitwise_and(o, _LOW),
                                       lax.shift_left(val, 16))
            buf_p[j, pl.ds(c * 16, 16)] = jnp.where(
                posbit16 == 0, lo_merged, hi_merged)
        return 0
    lax.fori_loop(0, 16, body, 0)


@functools.partial(
    pl.kernel,
    out_type=jax.ShapeDtypeStruct((_BH * _S, _D), jnp.bfloat16),
    mesh=_mesh,
    scratch_types=[
        pltpu.VMEM((_NBUF, _CS, _D), jnp.bfloat16),
        pltpu.VMEM((_Q, 16), jnp.int32),
        pltpu.VMEM((16,), jnp.int32),
        pltpu.VMEM((16,), jnp.int32),
        pltpu.VMEM((16, _D), jnp.int32),
        pltpu.VMEM((16, _D), jnp.int32),
        pltpu.SemaphoreType.DMA,
        pltpu.SemaphoreType.DMA,
        pltpu.SemaphoreType.DMA,
        pltpu.SemaphoreType.DMA,
        pltpu.SemaphoreType.DMA,
        pltpu.SemaphoreType.DMA,
        pltpu.SemaphoreType.DMA,
    ],
)
def _sc_update(pos_hbm, cache_hbm, vals_hbm, out_ref,
               ring, pos_b, idx_p, idx_u, buf_p, buf_u,
               sem_in0, sem_in1, sem_in2, sem_out0, sem_out1, sem_out2,
               sem_sc):
    sem_in = (sem_in0, sem_in1, sem_in2)
    sem_out = (sem_out0, sem_out1, sem_out2)
    cid = lax.axis_index("c")
    sid = lax.axis_index("s")
    wid = sid * 2 + cid
    pltpu.async_copy(pos_hbm, pos_b, sem_sc).wait()
    def chunk_rows(c):
        bh = wid * _BH_PW + c // (_S // _CS)
        s0 = (c % (_S // _CS)) * _CS
        return bh * _S + s0
    in_cp = {}
    out_cp = {}
    for c in range(_NBUF):
        in_cp[c] = pltpu.async_copy(
            cache_hbm.at[pl.ds(chunk_rows(c), _CS)], ring.at[c % _NBUF],
            sem_in[c % _NBUF])
    for c in range(_NCH):
        if c >= _NBUF:
            out_cp[c - _NBUF].wait()
            in_cp[c] = pltpu.async_copy(
                cache_hbm.at[pl.ds(chunk_rows(c), _CS)], ring.at[c % _NBUF],
                sem_in[c % _NBUF])
        in_cp[c].wait()
        out_cp[c] = pltpu.async_copy(
            ring.at[c % _NBUF], out_ref.at[pl.ds(chunk_rows(c), _CS)],
            sem_out[c % _NBUF])
    for c in range(_NCH - _NBUF, _NCH):
        out_cp[c].wait()
    lanes = lax.broadcasted_iota(jnp.int32, (16,), 0)
    bh_l = wid * _BH_PW + jnp.bitwise_and(lanes, 3)
    o32 = out_ref.bitcast(jnp.int32)
    u32 = vals_hbm.bitcast(jnp.int32)
    for q in range(_Q):
        pos16 = pos_b[q, :]
        posbit16 = jnp.bitwise_and(pos16, 1)
        idx_p[...] = bh_l * _SP + lax.shift_right_logical(pos16, 1)
        idx_u[...] = bh_l * (_Q // 2) + (q // 2)
        cp = pltpu.async_copy(o32.at[idx_p], buf_p, sem_sc)
        cu = pltpu.async_copy(u32.at[idx_u], buf_u, sem_sc)
        cp.wait()
        cu.wait()
        _merge_rows(buf_p, buf_u, posbit16, q % 2)
        pltpu.async_copy(buf_p, o32.at[idx_p], sem_sc).wait()


def kernel(k_cache, v_cache, input_pos, k_val, v_val):
    pos_b = jnp.broadcast_to(input_pos[:, None], (_Q, 16))
    V = _sc_update(pos_b, v_cache.reshape(_BH * _S, _D),
                   v_val.reshape(_BH * _Q, _D)).reshape(_B, _H, _S, _D)
    K = _tc_update(k_cache, k_val, input_pos)
    return (K, K, V)


# speedup vs baseline: 1.1578x; 1.1578x over previous
import jax
import jax.numpy as jnp
from jax.experimental import pallas as pl
from jax.experimental.pallas import tpu as pltpu

_B, _H, _S, _D, _Q = 8, 16, 4096, 128, 8
_BH = _B * _H
_BHB = 8
_W = 8


def _fused_kernel(pos_ref, c_ref, v_ref, o_ref):
    o_ref[...] = c_ref[...]
    sub_iota = jax.lax.broadcasted_iota(jnp.int32, (1, _W, 1), 1)
    for q in range(_Q):
        pos = pos_ref[q]
        wb = pl.multiple_of((pos // _W) * _W, _W)
        r = pos % _W
        mask = sub_iota == r
        o_ref[:, pl.ds(wb, _W), :] = jnp.where(
            mask, v_ref[:, q:q + 1, :], o_ref[:, pl.ds(wb, _W), :])


def _update(cache, vals, input_pos):
    c = cache.reshape(_BH, _S, _D)
    v = vals.reshape(_BH, _Q, _D)
    grid = (_BH // _BHB,)
    cache_spec = pl.BlockSpec((_BHB, _S, _D), lambda i: (i, 0, 0))
    val_spec = pl.BlockSpec((_BHB, _Q, _D), lambda i: (i, 0, 0))
    out = pl.pallas_call(
        _fused_kernel,
        grid=grid,
        in_specs=[
            pl.BlockSpec(memory_space=pltpu.SMEM),
            cache_spec, val_spec,
        ],
        out_specs=cache_spec,
        out_shape=jax.ShapeDtypeStruct((_BH, _S, _D), c.dtype),
    )(input_pos, c, v)
    return out.reshape(_B, _H, _S, _D)


def kernel(k_cache, v_cache, input_pos, k_val, v_val):
    K = _update(k_cache, k_val, input_pos)
    V = _update(v_cache, v_val, input_pos)
    return (K, K, V)
